# Initial kernel scaffold; baseline (speedup 1.0000x reference)
import jax
import jax.numpy as jnp
from jax.experimental import pallas as pl


def kernel(img_features, intrinsics, extrinsics, dn_w, dn_b, conv1_w, bn1_g, bn1_b, bn1_m, bn1_v, conv2_w, bn2_g, bn2_b, bn2_m, bn2_v):
    raise NotImplementedError("write your pallas kernel here")


# speedup vs baseline: 6.4065x; 6.4065x over previous
import functools

import jax
import jax.numpy as jnp
from jax import lax
from jax.experimental import pallas as pl
from jax.experimental.pallas import tpu as pltpu
from jax.experimental.pallas import tpu_sc as plsc

B, C, H, W = 4, 64, 32, 88
D = 80
OC = 64
XMIN, XRES = -51.2, 0.8
YMIN, YRES = -51.2, 0.8
BEVH, BEVW = 128, 128
DMIN, DMAX = 4.0, 45.0
HW = H * W
NPT = D * HW
NCELL = BEVH * BEVW
NPIX = NCELL

NSC = 2
NTILE = 16
HWT = HW // NTILE
GP = 8
NG = HWT // GP
RPG = GP * D
NROW = 128
NDMA = RPG // NROW
STRIPE = NCELL // NTILE


def _stage_a_body(imgT_ref, dnwT_ref, dnb_ref, valid_ref, ctx_ref, w_ref):
    x = imgT_ref[0]
    wmat = dnwT_ref[...]
    cd = jnp.dot(x, wmat, preferred_element_type=jnp.float32) + dnb_ref[...]
    ctx_ref[0] = cd[:, :C]
    logits = cd[:, C:]
    m = jnp.max(logits, axis=1, keepdims=True)
    e = jnp.exp(logits - m)
    sm = e / jnp.sum(e, axis=1, keepdims=True)
    w_ref[0] = sm * valid_ref[0]


def _stage_a(imgT, dn_wT, dn_b2, valid_pm):
    return pl.pallas_call(
        _stage_a_body,
        grid=(B,),
        in_specs=[
            pl.BlockSpec((1, HW, C), lambda b: (b, 0, 0)),
            pl.BlockSpec((C, D + C), lambda b: (0, 0)),
            pl.BlockSpec((1, D + C), lambda b: (0, 0)),
            pl.BlockSpec((1, HW, D), lambda b: (b, 0, 0)),
        ],
        out_specs=[
            pl.BlockSpec((1, HW, C), lambda b: (b, 0, 0)),
            pl.BlockSpec((1, HW, D), lambda b: (b, 0, 0)),
        ],
        out_shape=[
            jax.ShapeDtypeStruct((B, HW, C), jnp.float32),
            jax.ShapeDtypeStruct((B, HW, D), jnp.float32),
        ],
    )(imgT, dn_wT, dn_b2, valid_pm)


def _splat_body(ctx_hbm, w_hbm, idx_hbm, out_hbm,
                acc, ctx_v, w_v, idxf_v, idx_v, rows_v, zero_v):
    cid = lax.axis_index("c")
    sid = lax.axis_index("s")

    zv = jnp.zeros((16,), jnp.float32)

    def zero_row(i, _):
        for j in range(C // 16):
            zero_v[i, pl.ds(j * 16, 16)] = zv
        return 0

    lax.fori_loop(0, NROW, zero_row, 0)
    for q in range(STRIPE // NROW):
        pltpu.sync_copy(zero_v,
                        acc.at[pl.ds(sid * STRIPE + q * NROW, NROW)])
    plsc.subcore_barrier()

    for k in range(B // NSC):
        b = cid * (B // NSC) + k
        pltpu.sync_copy(ctx_hbm.at[pl.ds(b * HW + sid * HWT, HWT)], ctx_v)
        base_pt = (b * HW + sid * HWT) * D

        def group_body(g, _):
            gbase = base_pt + g * RPG
            pltpu.sync_copy(w_hbm.at[pl.ds(gbase, RPG)], w_v)
            pltpu.sync_copy(idx_hbm.at[pl.ds(gbase, RPG)], idxf_v)
            for j in range(NDMA):
                for l in range(NROW // 16):
                    idx_v[j, pl.ds(l * 16, 16)] = (
                        idxf_v[pl.ds(j * NROW + l * 16, 16)])

            def pix_body(p, _):
                crow = g * GP + p
                cvecs = [ctx_v[crow, pl.ds(j * 16, 16)]
                         for j in range(C // 16)]
                for dv in range(D // 16):
                    w16 = w_v[pl.ds(p * D + dv * 16, 16)]
                    for jj in range(16):
                        dd = dv * 16 + jj
                        wb = jnp.full((16,), w16[jj], jnp.float32)
                        for j in range(C // 16):
                            rows_v[p * D + dd, pl.ds(j * 16, 16)] = (
                                wb * cvecs[j])
                return 0

            lax.fori_loop(0, GP, pix_body, 0)
            for j in range(NDMA):
                pltpu.sync_copy(rows_v.at[pl.ds(j * NROW, NROW)],
                                acc.at[idx_v.at[j]], add=True)
            return 0

        lax.fori_loop(0, NG, group_body, 0)
        plsc.subcore_barrier()

        for q in range(2):
            half = STRIPE // 2
            off = sid * STRIPE + q * half
            pltpu.sync_copy(acc.at[pl.ds(off, half)],
                            rows_v.at[pl.ds(0, half)])
            pltpu.sync_copy(rows_v.at[pl.ds(0, half)],
                            out_hbm.at[pl.ds(b * NCELL + off, half)])
        if k == 0:
            plsc.subcore_barrier()
            for q in range(STRIPE // NROW):
                pltpu.sync_copy(zero_v,
                                acc.at[pl.ds(sid * STRIPE + q * NROW, NROW)])
            plsc.subcore_barrier()


def _stage_b(ctx_rows, w_flat, idx_flat):
    mesh = plsc.VectorSubcoreMesh(core_axis_name="c", subcore_axis_name="s")
    return pl.kernel(
        _splat_body,
        out_type=jax.ShapeDtypeStruct((B * NCELL, C), jnp.float32),
        mesh=mesh,
        scratch_types=[
            pltpu.VMEM_SHARED((NCELL, C), jnp.float32),
            pltpu.VMEM((HWT, C), jnp.float32),
            pltpu.VMEM((RPG,), jnp.float32),
            pltpu.VMEM((RPG,), jnp.int32),
            pltpu.VMEM((NDMA, NROW), jnp.int32),
            pltpu.VMEM((RPG, C), jnp.float32),
            pltpu.VMEM((NROW, C), jnp.float32),
        ],
        compiler_params=pltpu.CompilerParams(use_tc_tiling_on_sc=False),
    )(ctx_rows, w_flat, idx_flat)


CCH = 4096
NCH = NPIX // CCH
CM = 256


def _stage_c_body(prev_ref, cur_ref, next_ref, w_ref, b_ref, out_ref):
    ch = pl.program_id(1)
    first = (ch == 0).astype(jnp.float32)
    last = (ch == pl.num_programs(1) - 1).astype(jnp.float32)
    prevm = prev_ref[0, CCH - CM:, :] * (1.0 - first)
    nextm = next_ref[0, :CM, :] * (1.0 - last)
    X = jnp.concatenate([prevm, cur_ref[0], nextm], axis=0)
    col = lax.broadcasted_iota(jnp.int32, (CCH, 1), 0) % BEVW
    acc = jnp.zeros((CCH, OC), jnp.float32) + b_ref[...]
    for dy in range(3):
        for dx in range(3):
            o = (dy - 1) * BEVW + (dx - 1)
            Xs = X[CM + o:CM + o + CCH]
            if dx == 0:
                Xs = jnp.where(col == 0, 0.0, Xs)
            elif dx == 2:
                Xs = jnp.where(col == BEVW - 1, 0.0, Xs)
            acc = acc + jnp.dot(Xs, w_ref[dy, dx],
                                preferred_element_type=jnp.float32)
    out_ref[0] = jnp.maximum(acc, 0.0)


def _stage_c(bev, wf, bv):
    return pl.pallas_call(
        _stage_c_body,
        grid=(B, NCH),
        in_specs=[
            pl.BlockSpec((1, CCH, C),
                         lambda b, ch: (b, jnp.maximum(ch - 1, 0), 0)),
            pl.BlockSpec((1, CCH, C), lambda b, ch: (b, ch, 0)),
            pl.BlockSpec((1, CCH, C),
                         lambda b, ch: (b, jnp.minimum(ch + 1, NCH - 1), 0)),
            pl.BlockSpec((3, 3, C, OC), lambda b, ch: (0, 0, 0, 0)),
            pl.BlockSpec((1, OC), lambda b, ch: (0, 0)),
        ],
        out_specs=pl.BlockSpec((1, CCH, OC), lambda b, ch: (b, ch, 0)),
        out_shape=jax.ShapeDtypeStruct((B, NPIX, OC), jnp.float32),
    )(bev, bev, bev, wf, bv)


def kernel(img_features, intrinsics, extrinsics, dn_w, dn_b, conv1_w,
           bn1_g, bn1_b, bn1_m, bn1_v, conv2_w, bn2_g, bn2_b, bn2_m, bn2_v):
    imgT = img_features.reshape(B, C, HW).transpose(0, 2, 1)
    dn_wT = dn_w.T
    dn_b2 = dn_b.reshape(1, D + C)

    depths = jnp.linspace(DMIN, DMAX, D)
    ys, xs = jnp.meshgrid(jnp.arange(H, dtype=jnp.float32),
                          jnp.arange(W, dtype=jnp.float32), indexing='ij')
    pix = jnp.stack([xs, ys, jnp.ones_like(xs)], axis=0).reshape(3, HW)
    inv_K = jnp.linalg.inv(intrinsics)
    cam_unit = inv_K @ pix[None]
    cam_pts = depths.reshape(1, 1, D, 1) * cam_unit[:, :, None, :]
    hom = jnp.concatenate(
        [cam_pts, jnp.ones((B, 1, D, HW), cam_pts.dtype)], axis=1
    ).reshape(B, 4, NPT)
    ego = (extrinsics @ hom)[:, :3]
    xc = ((ego[:, 0] - XMIN) / XRES).astype(jnp.int32)
    yc = ((ego[:, 1] - YMIN) / YRES).astype(jnp.int32)
    valid = (xc >= 0) & (xc < BEVW) & (yc >= 0) & (yc < BEVH)
    idx = jnp.where(valid, yc * BEVW + xc, 0)
    valid_pm = valid.reshape(B, D, HW).transpose(0, 2, 1).astype(jnp.float32)
    idx_pm = idx.reshape(B, D, HW).transpose(0, 2, 1)

    ctx, w = _stage_a(imgT, dn_wT, dn_b2, valid_pm)

    w_flat = w.reshape(B * NPT)
    idx_flat = idx_pm.reshape(B * NPT)
    bev = _stage_b(ctx.reshape(B * HW, C), w_flat, idx_flat)
    bev = bev.reshape(B, NCELL, C)

    s1 = bn1_g / jnp.sqrt(bn1_v + 1e-5)
    t1 = bn1_b - bn1_m * s1
    w1f = conv1_w.transpose(2, 3, 1, 0) * s1
    s2 = bn2_g / jnp.sqrt(bn2_v + 1e-5)
    t2 = bn2_b - bn2_m * s2
    w2f = conv2_w.transpose(2, 3, 1, 0) * s2
    h1 = _stage_c(bev, w1f, t1.reshape(1, OC))
    out = _stage_c(h1, w2f, t2.reshape(1, OC))
    return out.reshape(B, BEVH, BEVW, OC).transpose(0, 3, 1, 2)


# speedup vs baseline: 6.4175x; 1.0017x over previous
import functools

import jax
import jax.numpy as jnp
from jax import lax
from jax.experimental import pallas as pl
from jax.experimental.pallas import tpu as pltpu
from jax.experimental.pallas import tpu_sc as plsc

B, C, H, W = 4, 64, 32, 88
D = 80
OC = 64
XMIN, XRES = -51.2, 0.8
YMIN, YRES = -51.2, 0.8
BEVH, BEVW = 128, 128
DMIN, DMAX = 4.0, 45.0
HW = H * W
NPT = D * HW
NCELL = BEVH * BEVW
NPIX = NCELL

NSC = 2
NTILE = 16
HWT = HW // NTILE
GP = 8
NG = HWT // GP
RPG = GP * D
NROW = 128
NDMA = RPG // NROW
STRIPE = NCELL // NTILE


def _stage_a_body(imgT_ref, dnwT_ref, dnb_ref, valid_ref, ctx_ref, w_ref):
    x = imgT_ref[0]
    wmat = dnwT_ref[...]
    cd = jnp.dot(x, wmat, preferred_element_type=jnp.float32) + dnb_ref[...]
    ctx_ref[0] = cd[:, :C]
    logits = cd[:, C:]
    m = jnp.max(logits, axis=1, keepdims=True)
    e = jnp.exp(logits - m)
    sm = e / jnp.sum(e, axis=1, keepdims=True)
    w_ref[0] = sm * valid_ref[0]


def _stage_a(imgT, dn_wT, dn_b2, valid_pm):
    return pl.pallas_call(
        _stage_a_body,
        grid=(B,),
        in_specs=[
            pl.BlockSpec((1, HW, C), lambda b: (b, 0, 0)),
            pl.BlockSpec((C, D + C), lambda b: (0, 0)),
            pl.BlockSpec((1, D + C), lambda b: (0, 0)),
            pl.BlockSpec((1, HW, D), lambda b: (b, 0, 0)),
        ],
        out_specs=[
            pl.BlockSpec((1, HW, C), lambda b: (b, 0, 0)),
            pl.BlockSpec((1, HW, D), lambda b: (b, 0, 0)),
        ],
        out_shape=[
            jax.ShapeDtypeStruct((B, HW, C), jnp.float32),
            jax.ShapeDtypeStruct((B, HW, D), jnp.float32),
        ],
    )(imgT, dn_wT, dn_b2, valid_pm)


def _splat_body(ctx_hbm, w_hbm, idx_hbm, out_hbm,
                acc, ctx_v, w_v, idxf_v, idx_v, rows_v, zero_v):
    cid = lax.axis_index("c")
    sid = lax.axis_index("s")

    zv = jnp.zeros((16,), jnp.float32)

    def zero_row(i, _):
        for j in range(C // 16):
            zero_v[i, pl.ds(j * 16, 16)] = zv
        return 0

    lax.fori_loop(0, NROW, zero_row, 0)
    for q in range(STRIPE // NROW):
        pltpu.sync_copy(zero_v,
                        acc.at[pl.ds(sid * STRIPE + q * NROW, NROW)])
    plsc.subcore_barrier()

    for k in range(B // NSC):
        b = cid * (B // NSC) + k
        pltpu.sync_copy(ctx_hbm.at[pl.ds(b * HW + sid * HWT, HWT)], ctx_v)
        base_pt = (b * HW + sid * HWT) * D

        def group_body(g, _):
            gbase = base_pt + g * RPG
            pltpu.sync_copy(w_hbm.at[pl.ds(gbase, RPG)], w_v)
            pltpu.sync_copy(idx_hbm.at[pl.ds(gbase, RPG)], idxf_v)
            for j in range(NDMA):
                for l in range(NROW // 16):
                    idx_v[j, pl.ds(l * 16, 16)] = (
                        idxf_v[pl.ds(j * NROW + l * 16, 16)])

            def pix_body(p, _):
                crow = g * GP + p
                cvecs = [ctx_v[crow, pl.ds(j * 16, 16)]
                         for j in range(C // 16)]
                for dv in range(D // 16):
                    w16 = w_v[pl.ds(p * D + dv * 16, 16)]
                    for jj in range(16):
                        dd = dv * 16 + jj
                        wb = jnp.full((16,), w16[jj], jnp.float32)
                        for j in range(C // 16):
                            rows_v[p * D + dd, pl.ds(j * 16, 16)] = (
                                wb * cvecs[j])
                return 0

            lax.fori_loop(0, GP, pix_body, 0)
            for j in range(NDMA):
                pltpu.sync_copy(rows_v.at[pl.ds(j * NROW, NROW)],
                                acc.at[idx_v.at[j]], add=True)
            return 0

        lax.fori_loop(0, NG, group_body, 0)
        plsc.subcore_barrier()

        for q in range(2):
            half = STRIPE // 2
            off = sid * STRIPE + q * half
            pltpu.sync_copy(acc.at[pl.ds(off, half)],
                            rows_v.at[pl.ds(0, half)])
            pltpu.sync_copy(rows_v.at[pl.ds(0, half)],
                            out_hbm.at[pl.ds(b * NCELL + off, half)])
        if k == 0:
            plsc.subcore_barrier()
            for q in range(STRIPE // NROW):
                pltpu.sync_copy(zero_v,
                                acc.at[pl.ds(sid * STRIPE + q * NROW, NROW)])
            plsc.subcore_barrier()


def _stage_b(ctx_rows, w_flat, idx_flat):
    mesh = plsc.VectorSubcoreMesh(core_axis_name="c", subcore_axis_name="s")
    return pl.kernel(
        _splat_body,
        out_type=jax.ShapeDtypeStruct((B * NCELL, C), jnp.float32),
        mesh=mesh,
        scratch_types=[
            pltpu.VMEM_SHARED((NCELL, C), jnp.float32),
            pltpu.VMEM((HWT, C), jnp.float32),
            pltpu.VMEM((RPG,), jnp.float32),
            pltpu.VMEM((RPG,), jnp.int32),
            pltpu.VMEM((NDMA, NROW), jnp.int32),
            pltpu.VMEM((RPG, C), jnp.float32),
            pltpu.VMEM((NROW, C), jnp.float32),
        ],
        compiler_params=pltpu.CompilerParams(use_tc_tiling_on_sc=False),
    )(ctx_rows, w_flat, idx_flat)


CCH = 4096
NCH = NPIX // CCH
CM = 256


def _stage_c_body(prev_ref, cur_ref, next_ref, w_ref, b_ref, out_ref):
    ch = pl.program_id(1)
    first = (ch == 0).astype(jnp.float32)
    last = (ch == pl.num_programs(1) - 1).astype(jnp.float32)
    prevm = prev_ref[0, CCH - CM:, :] * (1.0 - first)
    nextm = next_ref[0, :CM, :] * (1.0 - last)
    X = jnp.concatenate([prevm, cur_ref[0], nextm], axis=0)
    col = lax.broadcasted_iota(jnp.int32, (CCH, 1), 0) % BEVW
    acc = jnp.zeros((CCH, OC), jnp.float32) + b_ref[...]
    for dy in range(3):
        for dx in range(3):
            o = (dy - 1) * BEVW + (dx - 1)
            Xs = X[CM + o:CM + o + CCH]
            if dx == 0:
                Xs = jnp.where(col == 0, 0.0, Xs)
            elif dx == 2:
                Xs = jnp.where(col == BEVW - 1, 0.0, Xs)
            acc = acc + jnp.dot(Xs, w_ref[dy, dx],
                                preferred_element_type=jnp.float32)
    out_ref[0] = jnp.maximum(acc, 0.0)


def _stage_c(bev, wf, bv):
    return pl.pallas_call(
        _stage_c_body,
        grid=(B, NCH),
        in_specs=[
            pl.BlockSpec((1, CCH, C),
                         lambda b, ch: (b, jnp.maximum(ch - 1, 0), 0)),
            pl.BlockSpec((1, CCH, C), lambda b, ch: (b, ch, 0)),
            pl.BlockSpec((1, CCH, C),
                         lambda b, ch: (b, jnp.minimum(ch + 1, NCH - 1), 0)),
            pl.BlockSpec((3, 3, C, OC), lambda b, ch: (0, 0, 0, 0)),
            pl.BlockSpec((1, OC), lambda b, ch: (0, 0)),
        ],
        out_specs=pl.BlockSpec((1, CCH, OC), lambda b, ch: (b, ch, 0)),
        out_shape=jax.ShapeDtypeStruct((B, NPIX, OC), jnp.float32),
    )(bev, bev, bev, wf, bv)


def kernel(img_features, intrinsics, extrinsics, dn_w, dn_b, conv1_w,
           bn1_g, bn1_b, bn1_m, bn1_v, conv2_w, bn2_g, bn2_b, bn2_m, bn2_v):
    imgT = img_features.reshape(B, C, HW).transpose(0, 2, 1)
    dn_wT = dn_w.T
    dn_b2 = dn_b.reshape(1, D + C)

    depths = jnp.linspace(DMIN, DMAX, D)
    ys, xs = jnp.meshgrid(jnp.arange(H, dtype=jnp.float32),
                          jnp.arange(W, dtype=jnp.float32), indexing='ij')
    pix = jnp.stack([xs, ys, jnp.ones_like(xs)], axis=0).reshape(3, HW)
    inv_K = jnp.linalg.inv(intrinsics)
    cam_unit = inv_K @ pix[None]
    cam_pts = depths.reshape(1, 1, 1, D) * cam_unit[:, :, :, None]
    hom = jnp.concatenate(
        [cam_pts, jnp.ones((B, 1, HW, D), cam_pts.dtype)], axis=1
    ).reshape(B, 4, NPT)
    ego = (extrinsics @ hom)[:, :3]
    xc = ((ego[:, 0] - XMIN) / XRES).astype(jnp.int32)
    yc = ((ego[:, 1] - YMIN) / YRES).astype(jnp.int32)
    valid = (xc >= 0) & (xc < BEVW) & (yc >= 0) & (yc < BEVH)
    idx_pm = jnp.where(valid, yc * BEVW + xc, 0)
    valid_pm = valid.reshape(B, HW, D).astype(jnp.float32)

    ctx, w = _stage_a(imgT, dn_wT, dn_b2, valid_pm)

    w_flat = w.reshape(B * NPT)
    idx_flat = idx_pm.reshape(B * NPT)
    bev = _stage_b(ctx.reshape(B * HW, C), w_flat, idx_flat)
    bev = bev.reshape(B, NCELL, C)

    s1 = bn1_g / jnp.sqrt(bn1_v + 1e-5)
    t1 = bn1_b - bn1_m * s1
    w1f = conv1_w.transpose(2, 3, 1, 0) * s1
    s2 = bn2_g / jnp.sqrt(bn2_v + 1e-5)
    t2 = bn2_b - bn2_m * s2
    w2f = conv2_w.transpose(2, 3, 1, 0) * s2
    h1 = _stage_c(bev, w1f, t1.reshape(1, OC))
    out = _stage_c(h1, w2f, t2.reshape(1, OC))
    return out.reshape(B, BEVH, BEVW, OC).transpose(0, 3, 1, 2)
